# baseline (speedup 1.0000x reference)
import functools

import jax
import jax.numpy as jnp
from jax import lax
from jax.experimental import pallas as pl
from jax.experimental.pallas import tpu as pltpu
from jax.experimental.pallas import tpu_sc as plsc

B, F, L, V, D = 4096, 26, 20, 100000, 64

NC, NS, LANES = 2, 16, 16
NW = NC * NS
BAGS = B * F
BAGS_W = BAGS // NW
CH_BAGS = 32
N_CH = BAGS_W // CH_BAGS
ROWS = CH_BAGS * L
N_VREG = ROWS // LANES
GRP_PER_STEP = 10
BAG_PER_STEP = 8
N_STEP = CH_BAGS // BAG_PER_STEP


def _emb_body(tbl, idxf, dummy, out, idx0, idx1, gidx0, gidx1,
              rows0, rows1, outv0, outv1,
              gsem0, gsem1, osem0, osem1, isem0, isem1):
    idx_vs = (idx0, idx1)
    gidx_vs = (gidx0, gidx1)
    rows_vs = (rows0, rows1)
    out_vs = (outv0, outv1)
    gsems = (gsem0, gsem1)
    osems = (osem0, osem1)
    isems = (isem0, isem1)
    wid = lax.axis_index("s") * NC + lax.axis_index("c")
    wbase = wid * BAGS_W

    def idx_fire(c, p):
        base_bag = wbase + c * CH_BAGS
        pltpu.async_copy(
            idxf.at[pl.ds(base_bag * L, ROWS)], idx_vs[p], isems[p])

    def idx_wait(p):
        pltpu.make_async_copy(
            idxf.at[pl.ds(0, ROWS)], idx_vs[p], isems[p]).wait()

    def gidx(c, p):
        bvar_vec = lax.broadcast_in_dim(
            lax.rem(c * CH_BAGS, F), (LANES,), ())

        def vstep(i, carry):
            pos = lax.iota(jnp.int32, LANES) + lax.broadcast_in_dim(
                i * LANES, (LANES,), ())
            f = lax.rem(bvar_vec + lax.div(pos, L), F)
            gidx_vs[p][pl.ds(i * LANES, LANES)] = (
                idx_vs[p][pl.ds(i * LANES, LANES)] + f * V)
            return carry

        lax.fori_loop(0, N_VREG, vstep, 0)

    def enq_group(t, p1):
        gvec = gidx_vs[p1][pl.ds(t * LANES, LANES)]
        for k in range(LANES):
            pltpu.async_copy(
                tbl.at[gvec[k]],
                rows_vs[p1].at[t * (LANES // 2) + (k // 2),
                          pl.ds((k % 2) * D, D)],
                gsems[p1])

    def drain_rows(p):
        pltpu.make_async_copy(dummy, rows_vs[p], gsems[p]).wait()

    def acc_chunk(g, p, enq_p1):
        base_bag = wbase + g * CH_BAGS

        def mstep(m, carry):
            if enq_p1 is not None:
                for q in range(GRP_PER_STEP):
                    enq_group(m * GRP_PER_STEP + q, enq_p1)
            for jj in range(BAG_PER_STEP):
                j = m * BAG_PER_STEP + jj
                row0 = j * (L // 2)
                for col in range(D // LANES):
                    s = rows_vs[p][row0, pl.ds(col * LANES, LANES)]
                    for l in range(1, L):
                        s = s + rows_vs[p][row0 + l // 2,
                                       pl.ds((l % 2) * D + col * LANES,
                                             LANES)]
                    out_vs[p][j, pl.ds(col * LANES, LANES)] = s
            return carry

        lax.fori_loop(0, N_STEP, mstep, 0)
        pltpu.async_copy(out_vs[p], out.at[pl.ds(base_bag, CH_BAGS)],
                         osems[p])

    idx_fire(0, 0)
    idx_wait(0)
    gidx(0, 0)
    idx_fire(1, 1)

    def prol_enq(t, carry):
        enq_group(t, 0)
        return carry

    lax.fori_loop(0, N_VREG, prol_enq, 0)

    def pair(gg, carry):
        for sub in (0, 1):
            g = gg * 2 + sub
            p = sub

            @pl.when(g <= N_CH - 2)
            def _():
                idx_wait(1 - p)
                gidx(g + 1, 1 - p)

            @pl.when(g <= N_CH - 3)
            def _():
                idx_fire(g + 2, p)

            drain_rows(p)

            @pl.when(g >= 2)
            def _():
                pltpu.make_async_copy(
                    out_vs[p], out.at[pl.ds(0, CH_BAGS)], osems[p]).wait()

            acc_chunk(g, p, 1 - p)
        return carry

    lax.fori_loop(0, N_CH // 2, pair, 0)
    drain_rows(0)
    pltpu.make_async_copy(
        out_vs[0], out.at[pl.ds(0, CH_BAGS)], osems[0]).wait()
    pltpu.make_async_copy(
        out_vs[1], out.at[pl.ds(0, CH_BAGS)], osems[1]).wait()


@functools.partial(jax.jit, static_argnums=())
def _emb(tbl, idxf, dummy):
    mesh = plsc.VectorSubcoreMesh(core_axis_name="c", subcore_axis_name="s")
    run = pl.kernel(
        _emb_body,
        mesh=mesh,
        out_type=jax.ShapeDtypeStruct((BAGS, D), jnp.float32),
        scratch_types=[
            pltpu.VMEM((ROWS,), jnp.int32),
            pltpu.VMEM((ROWS,), jnp.int32),
            pltpu.VMEM((ROWS,), jnp.int32),
            pltpu.VMEM((ROWS,), jnp.int32),
            pltpu.VMEM((ROWS // 2, 2 * D), jnp.float32),
            pltpu.VMEM((ROWS // 2, 2 * D), jnp.float32),
            pltpu.VMEM((CH_BAGS, D), jnp.float32),
            pltpu.VMEM((CH_BAGS, D), jnp.float32),
            pltpu.SemaphoreType.DMA,
            pltpu.SemaphoreType.DMA,
            pltpu.SemaphoreType.DMA,
            pltpu.SemaphoreType.DMA,
            pltpu.SemaphoreType.DMA,
            pltpu.SemaphoreType.DMA,
        ],
    )
    return run(tbl, idxf, dummy)


def kernel(indices, tables):
    tbl = tables.reshape(F * V, D)
    idxf = indices.reshape(B * F * L)
    dummy = jnp.zeros((ROWS // 2, 2 * D), jnp.float32)
    out = _emb(tbl, idxf, dummy)
    return out.reshape(B, F * D)


# speedup vs baseline: 2.0244x; 2.0244x over previous
import functools

import jax
import jax.numpy as jnp
from jax import lax
from jax.experimental import pallas as pl
from jax.experimental.pallas import tpu as pltpu
from jax.experimental.pallas import tpu_sc as plsc

B, F, L, V, D = 4096, 26, 20, 100000, 64

NC, NS, LANES = 2, 16, 16
NW = NC * NS
BAGS = B * F
BAGS_W = BAGS // NW
CH_BAGS = 32
N_CH = BAGS_W // CH_BAGS
ROWS = CH_BAGS * L
N_VREG = ROWS // LANES
GRP_PER_STEP = 5
BAG_PER_STEP = 4
N_STEP = CH_BAGS // BAG_PER_STEP


def _emb_body(tbl, idxf, dummy, out, idx0, idx1, gidx0, gidx1,
              rows0, rows1, outv0, outv1,
              gsem0, gsem1, osem0, osem1, isem0, isem1):
    idx_vs = (idx0, idx1)
    gidx_vs = (gidx0, gidx1)
    rows_vs = (rows0, rows1)
    out_vs = (outv0, outv1)
    gsems = (gsem0, gsem1)
    osems = (osem0, osem1)
    isems = (isem0, isem1)
    wid = lax.axis_index("s") * NC + lax.axis_index("c")
    wbase = wid * BAGS_W

    def idx_fire(c, p):
        base_bag = wbase + c * CH_BAGS
        pltpu.async_copy(
            idxf.at[pl.ds(base_bag * L, ROWS)], idx_vs[p], isems[p])

    def idx_wait(p):
        pltpu.make_async_copy(
            idxf.at[pl.ds(0, ROWS)], idx_vs[p], isems[p]).wait()

    def gidx(c, p):
        bvar_vec = lax.broadcast_in_dim(
            lax.rem(c * CH_BAGS, F), (LANES,), ())

        def vstep(i, carry):
            pos = lax.iota(jnp.int32, LANES) + lax.broadcast_in_dim(
                i * LANES, (LANES,), ())
            f = lax.rem(bvar_vec + lax.div(pos, L), F)
            gidx_vs[p][pl.ds(i * LANES, LANES)] = (
                idx_vs[p][pl.ds(i * LANES, LANES)] + f * V)
            return carry

        lax.fori_loop(0, N_VREG, vstep, 0)

    def enq_group(t, p1):
        gvec = gidx_vs[p1][pl.ds(t * LANES, LANES)]
        for k in range(LANES):
            pltpu.async_copy(
                tbl.at[gvec[k]],
                rows_vs[p1].at[t * (LANES // 2) + (k // 2),
                          pl.ds((k % 2) * D, D)],
                gsems[p1])

    def drain_rows(p):
        pltpu.make_async_copy(dummy, rows_vs[p], gsems[p]).wait()

    def acc_chunk(g, p, enq_p1):
        base_bag = wbase + g * CH_BAGS

        def mstep(m, carry):
            if enq_p1 is not None:
                for q in range(GRP_PER_STEP):
                    enq_group(m * GRP_PER_STEP + q, enq_p1)
            for jj in range(BAG_PER_STEP):
                j = m * BAG_PER_STEP + jj
                row0 = j * (L // 2)
                for col in range(D // LANES):
                    s = rows_vs[p][row0, pl.ds(col * LANES, LANES)]
                    for l in range(1, L):
                        s = s + rows_vs[p][row0 + l // 2,
                                       pl.ds((l % 2) * D + col * LANES,
                                             LANES)]
                    out_vs[p][j, pl.ds(col * LANES, LANES)] = s
            return carry

        lax.fori_loop(0, N_STEP, mstep, 0)
        pltpu.async_copy(out_vs[p], out.at[pl.ds(base_bag, CH_BAGS)],
                         osems[p])

    idx_fire(0, 0)
    idx_wait(0)
    gidx(0, 0)
    idx_fire(1, 1)

    def prol_enq(t, carry):
        enq_group(t, 0)
        return carry

    lax.fori_loop(0, N_VREG, prol_enq, 0)

    def pair(gg, carry):
        for sub in (0, 1):
            g = gg * 2 + sub
            p = sub

            @pl.when(g <= N_CH - 2)
            def _():
                idx_wait(1 - p)
                gidx(g + 1, 1 - p)

            @pl.when(g <= N_CH - 3)
            def _():
                idx_fire(g + 2, p)

            drain_rows(p)

            @pl.when(g >= 2)
            def _():
                pltpu.make_async_copy(
                    out_vs[p], out.at[pl.ds(0, CH_BAGS)], osems[p]).wait()

            acc_chunk(g, p, 1 - p)
        return carry

    lax.fori_loop(0, N_CH // 2, pair, 0)
    drain_rows(0)
    pltpu.make_async_copy(
        out_vs[0], out.at[pl.ds(0, CH_BAGS)], osems[0]).wait()
    pltpu.make_async_copy(
        out_vs[1], out.at[pl.ds(0, CH_BAGS)], osems[1]).wait()


@functools.partial(jax.jit, static_argnums=())
def _emb(tbl, idxf, dummy):
    mesh = plsc.VectorSubcoreMesh(core_axis_name="c", subcore_axis_name="s")
    run = pl.kernel(
        _emb_body,
        mesh=mesh,
        out_type=jax.ShapeDtypeStruct((BAGS, D), jnp.float32),
        scratch_types=[
            pltpu.VMEM((ROWS,), jnp.int32),
            pltpu.VMEM((ROWS,), jnp.int32),
            pltpu.VMEM((ROWS,), jnp.int32),
            pltpu.VMEM((ROWS,), jnp.int32),
            pltpu.VMEM((ROWS // 2, 2 * D), jnp.float32),
            pltpu.VMEM((ROWS // 2, 2 * D), jnp.float32),
            pltpu.VMEM((CH_BAGS, D), jnp.float32),
            pltpu.VMEM((CH_BAGS, D), jnp.float32),
            pltpu.SemaphoreType.DMA,
            pltpu.SemaphoreType.DMA,
            pltpu.SemaphoreType.DMA,
            pltpu.SemaphoreType.DMA,
            pltpu.SemaphoreType.DMA,
            pltpu.SemaphoreType.DMA,
        ],
    )
    return run(tbl, idxf, dummy)


def kernel(indices, tables):
    tbl = tables.reshape(F * V, D)
    idxf = indices.reshape(B * F * L)
    dummy = jnp.zeros((ROWS // 2, 2 * D), jnp.float32)
    out = _emb(tbl, idxf, dummy)
    return out.reshape(B, F * D)


# speedup vs baseline: 2.1188x; 1.0466x over previous
import functools

import jax
import jax.numpy as jnp
from jax import lax
from jax.experimental import pallas as pl
from jax.experimental.pallas import tpu as pltpu
from jax.experimental.pallas import tpu_sc as plsc

B, F, L, V, D = 4096, 26, 20, 100000, 64

NC, NS, LANES = 2, 16, 16
NW = NC * NS
BAGS = B * F
BAGS_W = BAGS // NW
CH_BAGS = 32
N_CH = BAGS_W // CH_BAGS
ROWS = CH_BAGS * L
N_VREG = ROWS // LANES
GRP_PER_STEP = 5
BAG_PER_STEP = 2
N_STEP = CH_BAGS // BAG_PER_STEP


def _emb_body(tbl, idxf, dummy, out, idx0, idx1, gidx0, gidx1,
              rows0, rows1, outv0, outv1,
              gsem0, gsem1, osem0, osem1, isem0, isem1):
    idx_vs = (idx0, idx1)
    gidx_vs = (gidx0, gidx1)
    rows_vs = (rows0, rows1)
    out_vs = (outv0, outv1)
    gsems = (gsem0, gsem1)
    osems = (osem0, osem1)
    isems = (isem0, isem1)
    wid = lax.axis_index("s") * NC + lax.axis_index("c")
    wbase = wid * BAGS_W

    def idx_fire(c, p):
        base_bag = wbase + c * CH_BAGS
        pltpu.async_copy(
            idxf.at[pl.ds(base_bag * L, ROWS)], idx_vs[p], isems[p])

    def idx_wait(p):
        pltpu.make_async_copy(
            idxf.at[pl.ds(0, ROWS)], idx_vs[p], isems[p]).wait()

    def gidx(c, p):
        bvar_vec = lax.broadcast_in_dim(
            lax.rem(c * CH_BAGS, F), (LANES,), ())

        def vstep(i, carry):
            pos = lax.iota(jnp.int32, LANES) + lax.broadcast_in_dim(
                i * LANES, (LANES,), ())
            f = lax.rem(bvar_vec + lax.div(pos, L), F)
            gidx_vs[p][pl.ds(i * LANES, LANES)] = (
                idx_vs[p][pl.ds(i * LANES, LANES)] + f * V)
            return carry

        lax.fori_loop(0, N_VREG, vstep, 0)

    def enq_group(t, p1):
        gvec = gidx_vs[p1][pl.ds(t * LANES, LANES)]
        for k in range(LANES):
            pltpu.async_copy(
                tbl.at[gvec[k]],
                rows_vs[p1].at[t * (LANES // 2) + (k // 2),
                          pl.ds((k % 2) * D, D)],
                gsems[p1])

    def drain_rows(p):
        pltpu.make_async_copy(dummy, rows_vs[p], gsems[p]).wait()

    def acc_chunk(g, p, enq_p1):
        base_bag = wbase + g * CH_BAGS

        def mstep(m, carry):
            if enq_p1 is not None:
                for q in range(GRP_PER_STEP):
                    enq_group(m * GRP_PER_STEP + q, enq_p1)
            for jj in range(BAG_PER_STEP):
                j = m * BAG_PER_STEP + jj
                row0 = j * (L // 2)
                for col in range(D // LANES):
                    s = rows_vs[p][row0, pl.ds(col * LANES, LANES)]
                    for l in range(1, L):
                        s = s + rows_vs[p][row0 + l // 2,
                                       pl.ds((l % 2) * D + col * LANES,
                                             LANES)]
                    out_vs[p][j, pl.ds(col * LANES, LANES)] = s
            return carry

        lax.fori_loop(0, N_STEP, mstep, 0)
        pltpu.async_copy(out_vs[p], out.at[pl.ds(base_bag, CH_BAGS)],
                         osems[p])

    idx_fire(0, 0)
    idx_wait(0)
    gidx(0, 0)
    idx_fire(1, 1)

    def prol_enq(t, carry):
        enq_group(t, 0)
        return carry

    lax.fori_loop(0, N_VREG, prol_enq, 0)

    def pair(gg, carry):
        for sub in (0, 1):
            g = gg * 2 + sub
            p = sub

            @pl.when(g <= N_CH - 2)
            def _():
                idx_wait(1 - p)
                gidx(g + 1, 1 - p)

            @pl.when(g <= N_CH - 3)
            def _():
                idx_fire(g + 2, p)

            drain_rows(p)

            @pl.when(g >= 2)
            def _():
                pltpu.make_async_copy(
                    out_vs[p], out.at[pl.ds(0, CH_BAGS)], osems[p]).wait()

            def _enq(t, carry):
                enq_group(t, 1 - p)
                return carry

            lax.fori_loop(0, N_VREG, _enq, 0)
            acc_chunk(g, p, None)
        return carry

    lax.fori_loop(0, N_CH // 2, pair, 0)
    drain_rows(0)
    pltpu.make_async_copy(
        out_vs[0], out.at[pl.ds(0, CH_BAGS)], osems[0]).wait()
    pltpu.make_async_copy(
        out_vs[1], out.at[pl.ds(0, CH_BAGS)], osems[1]).wait()


@functools.partial(jax.jit, static_argnums=())
def _emb(tbl, idxf, dummy):
    mesh = plsc.VectorSubcoreMesh(core_axis_name="c", subcore_axis_name="s")
    run = pl.kernel(
        _emb_body,
        mesh=mesh,
        out_type=jax.ShapeDtypeStruct((BAGS, D), jnp.float32),
        scratch_types=[
            pltpu.VMEM((ROWS,), jnp.int32),
            pltpu.VMEM((ROWS,), jnp.int32),
            pltpu.VMEM((ROWS,), jnp.int32),
            pltpu.VMEM((ROWS,), jnp.int32),
            pltpu.VMEM((ROWS // 2, 2 * D), jnp.float32),
            pltpu.VMEM((ROWS // 2, 2 * D), jnp.float32),
            pltpu.VMEM((CH_BAGS, D), jnp.float32),
            pltpu.VMEM((CH_BAGS, D), jnp.float32),
            pltpu.SemaphoreType.DMA,
            pltpu.SemaphoreType.DMA,
            pltpu.SemaphoreType.DMA,
            pltpu.SemaphoreType.DMA,
            pltpu.SemaphoreType.DMA,
            pltpu.SemaphoreType.DMA,
        ],
    )
    return run(tbl, idxf, dummy)


def kernel(indices, tables):
    tbl = tables.reshape(F * V, D)
    idxf = indices.reshape(B * F * L)
    dummy = jnp.zeros((ROWS // 2, 2 * D), jnp.float32)
    out = _emb(tbl, idxf, dummy)
    return out.reshape(B, F * D)
